# baseline (speedup 1.0000x reference)
import functools

import jax
import jax.numpy as jnp
from jax import lax
from jax.experimental import pallas as pl
from jax.experimental.pallas import tpu as pltpu

B = 32
V = 1_000_000
CHUNK = 4096
NCHUNK = -(-V // CHUNK)

K1 = 0
K2 = 42
KS2 = K1 ^ K2 ^ 0x1BD11BDA

_ROT_A = (13, 15, 26, 6)
_ROT_B = (17, 29, 16, 24)


def _rotl(x, r):
    return lax.shift_left(x, jnp.uint32(r)) | lax.shift_right_logical(
        x, jnp.uint32(32 - r)
    )


def _rounds(x0, x1, rots):
    for r in rots:
        x0 = x0 + x1
        x1 = _rotl(x1, r)
        x1 = x0 ^ x1
    return x0, x1


def _threefry_bits(i):
    ks0 = jnp.uint32(K1)
    ks1 = jnp.uint32(K2)
    ks2 = jnp.uint32(KS2)
    x0 = jnp.full_like(i, ks0)
    x1 = i + ks1
    x0, x1 = _rounds(x0, x1, _ROT_A)
    x0 = x0 + ks1
    x1 = x1 + ks2 + jnp.uint32(1)
    x0, x1 = _rounds(x0, x1, _ROT_B)
    x0 = x0 + ks2
    x1 = x1 + ks0 + jnp.uint32(2)
    x0, x1 = _rounds(x0, x1, _ROT_A)
    x0 = x0 + ks0
    x1 = x1 + ks1 + jnp.uint32(3)
    x0, x1 = _rounds(x0, x1, _ROT_B)
    x0 = x0 + ks1
    x1 = x1 + ks2 + jnp.uint32(4)
    x0, x1 = _rounds(x0, x1, _ROT_A)
    x0 = x0 + ks2
    x1 = x1 + ks0 + jnp.uint32(5)
    return x0 ^ x1


import numpy as np

_TINY = np.float32(np.finfo(np.float32).tiny)
_NEG_INF = np.float32(-np.inf)


def _gumbel_from_bits(bits):
    fbits = lax.shift_right_logical(bits, jnp.uint32(9)) | jnp.uint32(0x3F800000)
    f = lax.bitcast_convert_type(fbits, jnp.float32) - jnp.float32(1.0)
    u = jnp.maximum(_TINY, f + _TINY)
    return -jnp.log(-jnp.log(u))


def _sample_kernel(x_ref, o_ref, acc_val, acc_idx):
    pid = pl.program_id(0)

    col = jax.lax.broadcasted_iota(jnp.uint32, (B, CHUNK), 1) + (
        pid.astype(jnp.uint32) * jnp.uint32(CHUNK)
    )
    row = jax.lax.broadcasted_iota(jnp.uint32, (B, CHUNK), 0)
    flat = row * jnp.uint32(V) + col

    bits = _threefry_bits(flat)
    g = _gumbel_from_bits(bits)
    v = x_ref[...] + g
    v = jnp.where(col < jnp.uint32(V), v, _NEG_INF)
    idx = col.astype(jnp.int32)

    @pl.when(pid == 0)
    def _init():
        acc_val[...] = v
        acc_idx[...] = idx

    @pl.when(pid != 0)
    def _update():
        better = v > acc_val[...]
        acc_val[...] = jnp.where(better, v, acc_val[...])
        acc_idx[...] = jnp.where(better, idx, acc_idx[...])

    @pl.when(pid == NCHUNK - 1)
    def _finish():
        av = acc_val[...]
        ai = acc_idx[...]
        m = jnp.max(av, axis=1, keepdims=True)
        cand = jnp.where(av == m, ai, jnp.int32(2**31 - 1))
        o_ref[0, :] = jnp.min(cand, axis=1)


@jax.jit
def kernel(logits):
    out = pl.pallas_call(
        _sample_kernel,
        grid=(NCHUNK,),
        in_specs=[pl.BlockSpec((B, CHUNK), lambda c: (0, c))],
        out_specs=pl.BlockSpec((1, B), lambda c: (0, 0)),
        out_shape=jax.ShapeDtypeStruct((1, B), jnp.int32),
        scratch_shapes=[
            pltpu.VMEM((B, CHUNK), jnp.float32),
            pltpu.VMEM((B, CHUNK), jnp.int32),
        ],
        compiler_params=pltpu.CompilerParams(
            dimension_semantics=("arbitrary",),
        ),
    )(logits)
    return out[0]


# speedup vs baseline: 1.6462x; 1.6462x over previous
import functools

import jax
import jax.numpy as jnp
from jax import lax
from jax.experimental import pallas as pl
from jax.experimental.pallas import tpu as pltpu

B = 32
V = 1_000_000
CHUNK = 4096
NCHUNK = -(-V // CHUNK)

K1 = 0
K2 = 42
KS2 = K1 ^ K2 ^ 0x1BD11BDA

_ROT_A = (13, 15, 26, 6)
_ROT_B = (17, 29, 16, 24)


def _rotl(x, r):
    return lax.shift_left(x, jnp.uint32(r)) | lax.shift_right_logical(
        x, jnp.uint32(32 - r)
    )


def _rounds(x0, x1, rots):
    for r in rots:
        x0 = x0 + x1
        x1 = _rotl(x1, r)
        x1 = x0 ^ x1
    return x0, x1


def _threefry_bits(i):
    ks0 = jnp.uint32(K1)
    ks1 = jnp.uint32(K2)
    ks2 = jnp.uint32(KS2)
    x0 = jnp.full_like(i, ks0)
    x1 = i + ks1
    x0, x1 = _rounds(x0, x1, _ROT_A)
    x0 = x0 + ks1
    x1 = x1 + ks2 + jnp.uint32(1)
    x0, x1 = _rounds(x0, x1, _ROT_B)
    x0 = x0 + ks2
    x1 = x1 + ks0 + jnp.uint32(2)
    x0, x1 = _rounds(x0, x1, _ROT_A)
    x0 = x0 + ks0
    x1 = x1 + ks1 + jnp.uint32(3)
    x0, x1 = _rounds(x0, x1, _ROT_B)
    x0 = x0 + ks1
    x1 = x1 + ks2 + jnp.uint32(4)
    x0, x1 = _rounds(x0, x1, _ROT_A)
    x0 = x0 + ks2
    x1 = x1 + ks0 + jnp.uint32(5)
    return x0 ^ x1


import numpy as np

_TINY = np.float32(np.finfo(np.float32).tiny)
_NEG_INF = np.float32(-np.inf)


def _gumbel_from_bits(bits):
    fbits = lax.shift_right_logical(bits, jnp.uint32(9)) | jnp.uint32(0x3F800000)
    f = lax.bitcast_convert_type(fbits, jnp.float32) - jnp.float32(1.0)
    u = jnp.maximum(_TINY, f + _TINY)
    return -jnp.log(-jnp.log(u))


TILE = 512
NTILE = CHUNK // TILE


def _sample_kernel(x_ref, o_ref, acc_val, acc_idx):
    pid = pl.program_id(0)

    base = jax.lax.broadcasted_iota(jnp.uint32, (B, TILE), 0) * jnp.uint32(V) + (
        jax.lax.broadcasted_iota(jnp.uint32, (B, TILE), 1)
        + pid.astype(jnp.uint32) * jnp.uint32(CHUNK)
    )
    col0 = pid * jnp.int32(CHUNK)

    for t in range(NTILE):
        flat = base + jnp.uint32(t * TILE)
        bits = _threefry_bits(flat)
        g = _gumbel_from_bits(bits)
        v = x_ref[:, pl.ds(t * TILE, TILE)] + g
        col = jax.lax.broadcasted_iota(jnp.int32, (B, TILE), 1) + (
            col0 + jnp.int32(t * TILE)
        )
        v = jnp.where(col < jnp.int32(V), v, _NEG_INF)

        if t == 0:
            is_first = pid == 0

            @pl.when(is_first)
            def _init():
                acc_val[...] = v
                acc_idx[...] = col

            @pl.when(jnp.logical_not(is_first))
            def _update():
                better = v > acc_val[...]
                acc_val[...] = jnp.where(better, v, acc_val[...])
                acc_idx[...] = jnp.where(better, col, acc_idx[...])
        else:
            better = v > acc_val[...]
            acc_val[...] = jnp.where(better, v, acc_val[...])
            acc_idx[...] = jnp.where(better, col, acc_idx[...])

    @pl.when(pid == NCHUNK - 1)
    def _finish():
        av = acc_val[...]
        ai = acc_idx[...]
        m = jnp.max(av, axis=1, keepdims=True)
        cand = jnp.where(av == m, ai, jnp.int32(2**31 - 1))
        o_ref[0, :] = jnp.min(cand, axis=1)


@jax.jit
def kernel(logits):
    out = pl.pallas_call(
        _sample_kernel,
        grid=(NCHUNK,),
        in_specs=[pl.BlockSpec((B, CHUNK), lambda c: (0, c))],
        out_specs=pl.BlockSpec((1, B), lambda c: (0, 0)),
        out_shape=jax.ShapeDtypeStruct((1, B), jnp.int32),
        scratch_shapes=[
            pltpu.VMEM((B, TILE), jnp.float32),
            pltpu.VMEM((B, TILE), jnp.int32),
        ],
        compiler_params=pltpu.CompilerParams(
            dimension_semantics=("arbitrary",),
        ),
    )(logits)
    return out[0]
